# baseline (speedup 1.0000x reference)
import functools

import jax
import jax.numpy as jnp
from jax import lax
from jax.experimental import pallas as pl
from jax.experimental.pallas import tpu as pltpu
from jax.experimental.pallas import tpu_sc as plsc

N = 10000
E = 320000
D_OUT = 64
D_HALF = D_OUT // 2
K_PROP = 10
ALPHA = 0.1

NC = 2
NS = 16
NW = NC * NS
LANES = 16

N_PAD = 10240
RPT = N_PAD // NS
CH = 256
NCH = 80
E_PAD = NS * NCH * CH
E_W = E_PAD // NW
NBUF = 4
ZR = 32
NGRP = NCH // NBUF
RB = 128
NRO = RPT // RB

_mesh = plsc.VectorSubcoreMesh(core_axis_name="c", subcore_axis_name="s")


@functools.partial(
    pl.kernel,
    out_type=jax.ShapeDtypeStruct((NC, N_PAD), jnp.float32),
    mesh=_mesh,
    compiler_params=pltpu.CompilerParams(needs_layout_passes=False),
    scratch_types=[
        pltpu.VMEM((E_W,), jnp.int32),
        pltpu.VMEM((N_PAD,), jnp.float32),
        pltpu.VMEM((RPT,), jnp.float32),
        pltpu.VMEM((RPT,), jnp.float32),
        pltpu.VMEM_SHARED((NS, N_PAD), jnp.float32),
    ],
)
def _deg_kernel(dst_hbm, degp_hbm, dst_v, hist_v, acc_v, ld_v, sh):
    cid = lax.axis_index("c")
    sid = lax.axis_index("s")
    wid = sid * NC + cid
    pltpu.sync_copy(dst_hbm.at[wid], dst_v)

    z = jnp.zeros((LANES,), jnp.float32)
    ones = jnp.ones((LANES,), jnp.float32)

    def zero_body(i, c):
        hist_v[pl.ds(i * LANES, LANES)] = z
        return c

    lax.fori_loop(0, N_PAD // LANES, zero_body, 0)

    def hist_body(i, c):
        idx = dst_v[pl.ds(i * LANES, LANES)]
        plsc.addupdate_scatter(hist_v, [idx], ones)
        return c

    lax.fori_loop(0, E_W // LANES, hist_body, 0)

    pltpu.sync_copy(hist_v, sh.at[sid])
    plsc.subcore_barrier()

    base = sid * RPT
    pltpu.sync_copy(sh.at[0, pl.ds(base, RPT)], acc_v)
    for j in range(1, NS):
        pltpu.sync_copy(sh.at[j, pl.ds(base, RPT)], ld_v)

        def add_body(i, c):
            s = pl.ds(i * LANES, LANES)
            acc_v[s] = acc_v[s] + ld_v[s]
            return c

        lax.fori_loop(0, RPT // LANES, add_body, 0)
    pltpu.sync_copy(acc_v, degp_hbm.at[cid, pl.ds(base, RPT)])


def _mlp_kernel(f_ref, w1_ref, b1_ref, w2_ref, b2_ref, degp_ref,
                h0s_ref, y0s_ref, nrep_ref):
    h = jnp.dot(f_ref[...], w1_ref[...], preferred_element_type=jnp.float32)
    h = jnp.maximum(h + b1_ref[...][None, :], 0.0)
    h = jnp.dot(h, w2_ref[...], preferred_element_type=jnp.float32)
    h = h + b2_ref[...][None, :]
    h0p = jnp.concatenate(
        [h, jnp.zeros((N_PAD - N, D_OUT), jnp.float32)], axis=0)
    deg = degp_ref[0, :] + degp_ref[1, :]
    nrm = lax.rsqrt(jnp.maximum(deg, 1.0))
    nrep_ref[...] = jnp.broadcast_to(nrm[:, None], (N_PAD, LANES))
    h0s_ref[0] = h0p[:, :D_HALF]
    h0s_ref[1] = h0p[:, D_HALF:]
    y0 = h0p * nrm[:, None]
    y0s_ref[0] = y0[:, :D_HALF]
    y0s_ref[1] = y0[:, D_HALF:]


_mlp_call = pl.pallas_call(
    _mlp_kernel,
    out_shape=(
        jax.ShapeDtypeStruct((NC, N_PAD, D_HALF), jnp.float32),
        jax.ShapeDtypeStruct((NC, N_PAD, D_HALF), jnp.float32),
        jax.ShapeDtypeStruct((N_PAD, LANES), jnp.float32),
    ),
)


@functools.partial(
    pl.kernel,
    out_type=jax.ShapeDtypeStruct((NC, N_PAD, D_HALF), jnp.float32),
    mesh=_mesh,
    compiler_params=pltpu.CompilerParams(use_tc_tiling_on_sc=False),
    scratch_types=[
        pltpu.VMEM((NCH, CH), jnp.int32),
        pltpu.VMEM((NCH, CH), jnp.int32),
        [pltpu.VMEM((CH, D_HALF), jnp.float32) for _ in range(NBUF)],
        pltpu.VMEM((ZR, D_HALF), jnp.float32),
        pltpu.VMEM((RPT, LANES), jnp.float32),
        pltpu.VMEM_SHARED((N_PAD, D_HALF), jnp.float32),
        pltpu.VMEM_SHARED((N_PAD, D_HALF), jnp.float32),
        [pltpu.SemaphoreType.DMA for _ in range(NBUF)],
        [pltpu.SemaphoreType.DMA for _ in range(NBUF)],
    ],
)
def _fused_prop(y0_hbm, h0s_hbm, nrep_hbm, src_hbm, dst_hbm, hs_hbm,
                src_v, dst_v, rows, zb, nrep_v, y_sh, agg_sh,
                gsem, ssem):
    cid = lax.axis_index("c")
    sid = lax.axis_index("s")
    base = sid * RPT
    pltpu.sync_copy(src_hbm.at[sid], src_v)
    pltpu.sync_copy(dst_hbm.at[sid], dst_v)
    pltpu.sync_copy(nrep_hbm.at[pl.ds(base, RPT)], nrep_v)

    z = jnp.zeros((LANES,), jnp.float32)

    def zb_body(r, c):
        zb[r, pl.ds(0, LANES)] = z
        zb[r, pl.ds(LANES, LANES)] = z
        return c

    lax.fori_loop(0, ZR, zb_body, 0)
    for zq in range(RPT // ZR):
        pltpu.sync_copy(zb, agg_sh.at[pl.ds(base + zq * ZR, ZR)])
    for q in range(NRO):
        s = pl.ds(base + q * RB, RB)
        pltpu.sync_copy(y0_hbm.at[cid, s], rows[0].at[pl.ds(0, RB)])
        pltpu.sync_copy(rows[0].at[pl.ds(0, RB)], y_sh.at[s])
    plsc.subcore_barrier()

    def _stream_edges():
        for b in range(NBUF):
            pltpu.async_copy(y_sh.at[src_v.at[b]], rows[b], gsem[b])

        def group_body(g, c2):
            for b in range(NBUF):
                k = g * NBUF + b
                pltpu.make_async_copy(y_sh.at[src_v.at[k]], rows[b],
                                      gsem[b]).wait()
                pltpu.async_copy(rows[b], agg_sh.at[dst_v.at[k]], ssem[b],
                                 add=True)
            for b in range(NBUF):
                k = g * NBUF + b
                pltpu.make_async_copy(rows[b], agg_sh.at[dst_v.at[k]],
                                      ssem[b]).wait()
                pltpu.async_copy(y_sh.at[src_v.at[k + NBUF]], rows[b],
                                 gsem[b])
            return c2

        lax.fori_loop(0, NGRP - 1, group_body, 0)

        for b in range(NBUF):
            k = (NGRP - 1) * NBUF + b
            pltpu.make_async_copy(y_sh.at[src_v.at[k]], rows[b],
                                  gsem[b]).wait()
            pltpu.async_copy(rows[b], agg_sh.at[dst_v.at[k]], ssem[b],
                             add=True)
        for b in range(NBUF):
            k = (NGRP - 1) * NBUF + b
            pltpu.make_async_copy(rows[b], agg_sh.at[dst_v.at[k]],
                                  ssem[b]).wait()
        plsc.subcore_barrier()

    def step_body(t, carry):
        _stream_edges()

        for q in range(NRO):
            s = pl.ds(base + q * RB, RB)
            pltpu.sync_copy(agg_sh.at[s], rows[0].at[pl.ds(0, RB)])
            pltpu.sync_copy(y0_hbm.at[cid, s], rows[1].at[pl.ds(0, RB)])

            def row_body(r, c2):
                nb = nrep_v[q * RB + r, pl.ds(0, LANES)]
                n2 = nb * nb
                for w in range(D_HALF // LANES):
                    sl = pl.ds(w * LANES, LANES)
                    a = rows[0][r, sl]
                    y0v = rows[1][r, sl]
                    rows[0][r, sl] = (1.0 - ALPHA) * (a * n2) + ALPHA * y0v
                return c2

            lax.fori_loop(0, RB, row_body, 0)
            pltpu.sync_copy(rows[0].at[pl.ds(0, RB)], y_sh.at[s])
            for zq in range(RB // ZR):
                pltpu.sync_copy(zb, agg_sh.at[pl.ds(base + q * RB + zq * ZR,
                                                    ZR)])
        plsc.subcore_barrier()
        return carry

    lax.fori_loop(0, K_PROP - 1, step_body, 0)

    _stream_edges()
    for q in range(NRO):
        s = pl.ds(base + q * RB, RB)
        pltpu.sync_copy(agg_sh.at[s], rows[0].at[pl.ds(0, RB)])
        pltpu.sync_copy(h0s_hbm.at[cid, s], rows[1].at[pl.ds(0, RB)])

        def fin_body(r, c2):
            nb = nrep_v[q * RB + r, pl.ds(0, LANES)]
            for w in range(D_HALF // LANES):
                sl = pl.ds(w * LANES, LANES)
                a = rows[0][r, sl]
                h0v = rows[1][r, sl]
                rows[1][r, sl] = (1.0 - ALPHA) * (a * nb) + ALPHA * h0v
            return c2

        lax.fori_loop(0, RB, fin_body, 0)
        pltpu.sync_copy(rows[1].at[pl.ds(0, RB)], hs_hbm.at[cid, s])


def kernel(features, edge_index, W1, b1, W2, b2):
    src = edge_index[0]
    dst = edge_index[1]
    pad = E_PAD - E
    src_p = jnp.concatenate([src, jnp.zeros((pad,), jnp.int32)])
    dst_p = jnp.concatenate([dst, jnp.full((pad,), N, jnp.int32)])
    src3 = src_p.reshape(NS, NCH, CH)
    dst3 = dst_p.reshape(NS, NCH, CH)
    dst2 = dst_p.reshape(NW, E_W)

    degp = _deg_kernel(dst2)
    h0s, y0s, nrep = _mlp_call(features, W1, b1, W2, b2, degp)
    hs = _fused_prop(y0s, h0s, nrep, src3, dst3)
    return jnp.concatenate([hs[0], hs[1]], axis=1)[:N]


# speedup vs baseline: 1.1082x; 1.1082x over previous
import functools

import jax
import jax.numpy as jnp
from jax import lax
from jax.experimental import pallas as pl
from jax.experimental.pallas import tpu as pltpu
from jax.experimental.pallas import tpu_sc as plsc

N = 10000
E = 320000
D_OUT = 64
D_HALF = D_OUT // 2
K_PROP = 10
ALPHA = 0.1

NC = 2
NS = 16
NW = NC * NS
LANES = 16

N_PAD = 10240
RPT = N_PAD // NS
CH = 128
NCH = 160
E_PAD = NS * NCH * CH
E_W = E_PAD // NW
NBUF = 8
ZR = 32
NGRP = NCH // NBUF
RB = 128
NRO = RPT // RB

_mesh = plsc.VectorSubcoreMesh(core_axis_name="c", subcore_axis_name="s")


@functools.partial(
    pl.kernel,
    out_type=jax.ShapeDtypeStruct((NC, N_PAD), jnp.float32),
    mesh=_mesh,
    compiler_params=pltpu.CompilerParams(needs_layout_passes=False),
    scratch_types=[
        pltpu.VMEM((E_W,), jnp.int32),
        pltpu.VMEM((N_PAD,), jnp.float32),
        pltpu.VMEM((RPT,), jnp.float32),
        pltpu.VMEM((RPT,), jnp.float32),
        pltpu.VMEM_SHARED((NS, N_PAD), jnp.float32),
    ],
)
def _deg_kernel(dst_hbm, degp_hbm, dst_v, hist_v, acc_v, ld_v, sh):
    cid = lax.axis_index("c")
    sid = lax.axis_index("s")
    wid = sid * NC + cid
    pltpu.sync_copy(dst_hbm.at[wid], dst_v)

    z = jnp.zeros((LANES,), jnp.float32)
    ones = jnp.ones((LANES,), jnp.float32)

    def zero_body(i, c):
        hist_v[pl.ds(i * LANES, LANES)] = z
        return c

    lax.fori_loop(0, N_PAD // LANES, zero_body, 0)

    def hist_body(i, c):
        idx = dst_v[pl.ds(i * LANES, LANES)]
        plsc.addupdate_scatter(hist_v, [idx], ones)
        return c

    lax.fori_loop(0, E_W // LANES, hist_body, 0)

    pltpu.sync_copy(hist_v, sh.at[sid])
    plsc.subcore_barrier()

    base = sid * RPT
    pltpu.sync_copy(sh.at[0, pl.ds(base, RPT)], acc_v)
    for j in range(1, NS):
        pltpu.sync_copy(sh.at[j, pl.ds(base, RPT)], ld_v)

        def add_body(i, c):
            s = pl.ds(i * LANES, LANES)
            acc_v[s] = acc_v[s] + ld_v[s]
            return c

        lax.fori_loop(0, RPT // LANES, add_body, 0)
    pltpu.sync_copy(acc_v, degp_hbm.at[cid, pl.ds(base, RPT)])


def _mlp_kernel(f_ref, w1_ref, b1_ref, w2_ref, b2_ref, degp_ref,
                h0s_ref, y0s_ref, nrep_ref):
    h = jnp.dot(f_ref[...], w1_ref[...], preferred_element_type=jnp.float32)
    h = jnp.maximum(h + b1_ref[...][None, :], 0.0)
    h = jnp.dot(h, w2_ref[...], preferred_element_type=jnp.float32)
    h = h + b2_ref[...][None, :]
    h0p = jnp.concatenate(
        [h, jnp.zeros((N_PAD - N, D_OUT), jnp.float32)], axis=0)
    deg = degp_ref[0, :] + degp_ref[1, :]
    nrm = lax.rsqrt(jnp.maximum(deg, 1.0))
    nrep_ref[...] = jnp.broadcast_to(nrm[:, None], (N_PAD, LANES))
    h0s_ref[0] = h0p[:, :D_HALF]
    h0s_ref[1] = h0p[:, D_HALF:]
    y0 = h0p * nrm[:, None]
    y0s_ref[0] = y0[:, :D_HALF]
    y0s_ref[1] = y0[:, D_HALF:]


_mlp_call = pl.pallas_call(
    _mlp_kernel,
    out_shape=(
        jax.ShapeDtypeStruct((NC, N_PAD, D_HALF), jnp.float32),
        jax.ShapeDtypeStruct((NC, N_PAD, D_HALF), jnp.float32),
        jax.ShapeDtypeStruct((N_PAD, LANES), jnp.float32),
    ),
)


@functools.partial(
    pl.kernel,
    out_type=jax.ShapeDtypeStruct((NC, N_PAD, D_HALF), jnp.float32),
    mesh=_mesh,
    compiler_params=pltpu.CompilerParams(use_tc_tiling_on_sc=False),
    scratch_types=[
        pltpu.VMEM((NCH, CH), jnp.int32),
        pltpu.VMEM((NCH, CH), jnp.int32),
        [pltpu.VMEM((CH, D_HALF), jnp.float32) for _ in range(NBUF)],
        pltpu.VMEM((ZR, D_HALF), jnp.float32),
        pltpu.VMEM((RPT, LANES), jnp.float32),
        pltpu.VMEM_SHARED((N_PAD, D_HALF), jnp.float32),
        pltpu.VMEM_SHARED((N_PAD, D_HALF), jnp.float32),
        [pltpu.SemaphoreType.DMA for _ in range(NBUF)],
        [pltpu.SemaphoreType.DMA for _ in range(NBUF)],
    ],
)
def _fused_prop(y0_hbm, h0s_hbm, nrep_hbm, src_hbm, dst_hbm, hs_hbm,
                src_v, dst_v, rows, zb, nrep_v, y_sh, agg_sh,
                gsem, ssem):
    cid = lax.axis_index("c")
    sid = lax.axis_index("s")
    base = sid * RPT
    pltpu.sync_copy(src_hbm.at[sid], src_v)
    pltpu.sync_copy(dst_hbm.at[sid], dst_v)
    pltpu.sync_copy(nrep_hbm.at[pl.ds(base, RPT)], nrep_v)

    z = jnp.zeros((LANES,), jnp.float32)

    def zb_body(r, c):
        zb[r, pl.ds(0, LANES)] = z
        zb[r, pl.ds(LANES, LANES)] = z
        return c

    lax.fori_loop(0, ZR, zb_body, 0)
    for zq in range(RPT // ZR):
        pltpu.sync_copy(zb, agg_sh.at[pl.ds(base + zq * ZR, ZR)])
    for q in range(NRO):
        s = pl.ds(base + q * RB, RB)
        pltpu.sync_copy(y0_hbm.at[cid, s], rows[0].at[pl.ds(0, RB)])
        pltpu.sync_copy(rows[0].at[pl.ds(0, RB)], y_sh.at[s])
    plsc.subcore_barrier()

    def _stream_edges():
        for b in range(NBUF):
            pltpu.async_copy(y_sh.at[src_v.at[b]], rows[b], gsem[b])

        def group_body(g, c2):
            for b in range(NBUF):
                k = g * NBUF + b
                pltpu.make_async_copy(y_sh.at[src_v.at[k]], rows[b],
                                      gsem[b]).wait()
                pltpu.async_copy(rows[b], agg_sh.at[dst_v.at[k]], ssem[b],
                                 add=True)
            for b in range(NBUF):
                k = g * NBUF + b
                pltpu.make_async_copy(rows[b], agg_sh.at[dst_v.at[k]],
                                      ssem[b]).wait()
                pltpu.async_copy(y_sh.at[src_v.at[k + NBUF]], rows[b],
                                 gsem[b])
            return c2

        lax.fori_loop(0, NGRP - 1, group_body, 0)

        for b in range(NBUF):
            k = (NGRP - 1) * NBUF + b
            pltpu.make_async_copy(y_sh.at[src_v.at[k]], rows[b],
                                  gsem[b]).wait()
            pltpu.async_copy(rows[b], agg_sh.at[dst_v.at[k]], ssem[b],
                             add=True)
        for b in range(NBUF):
            k = (NGRP - 1) * NBUF + b
            pltpu.make_async_copy(rows[b], agg_sh.at[dst_v.at[k]],
                                  ssem[b]).wait()
        plsc.subcore_barrier()

    def step_body(t, carry):
        _stream_edges()

        for q in range(NRO):
            pltpu.async_copy(y0_hbm.at[cid, pl.ds(base + q * RB, RB)],
                             rows[3 + q], gsem[q])
        for q in range(NRO):
            s = pl.ds(base + q * RB, RB)
            pltpu.sync_copy(agg_sh.at[s], rows[0])
            pltpu.make_async_copy(y0_hbm.at[cid, s], rows[3 + q],
                                  gsem[q]).wait()
            y0b = rows[3 + q]

            def row_body(r, c2):
                nb = nrep_v[q * RB + r, pl.ds(0, LANES)]
                n2 = nb * nb
                for w in range(D_HALF // LANES):
                    sl = pl.ds(w * LANES, LANES)
                    a = rows[0][r, sl]
                    y0v = y0b[r, sl]
                    rows[0][r, sl] = (1.0 - ALPHA) * (a * n2) + ALPHA * y0v
                return c2

            lax.fori_loop(0, RB, row_body, 0)
            pltpu.sync_copy(rows[0], y_sh.at[s])
            if q > 0:
                for zq in range(RB // ZR):
                    pltpu.make_async_copy(
                        zb,
                        agg_sh.at[pl.ds(base + (q - 1) * RB + zq * ZR, ZR)],
                        ssem[zq]).wait()
            for zq in range(RB // ZR):
                pltpu.async_copy(zb,
                                 agg_sh.at[pl.ds(base + q * RB + zq * ZR,
                                                 ZR)],
                                 ssem[zq])
        for zq in range(RB // ZR):
            pltpu.make_async_copy(
                zb, agg_sh.at[pl.ds(base + (NRO - 1) * RB + zq * ZR, ZR)],
                ssem[zq]).wait()
        plsc.subcore_barrier()
        return carry

    lax.fori_loop(0, K_PROP - 1, step_body, 0)

    _stream_edges()
    for q in range(NRO):
        s = pl.ds(base + q * RB, RB)
        pltpu.sync_copy(agg_sh.at[s], rows[0].at[pl.ds(0, RB)])
        pltpu.sync_copy(h0s_hbm.at[cid, s], rows[1].at[pl.ds(0, RB)])

        def fin_body(r, c2):
            nb = nrep_v[q * RB + r, pl.ds(0, LANES)]
            for w in range(D_HALF // LANES):
                sl = pl.ds(w * LANES, LANES)
                a = rows[0][r, sl]
                h0v = rows[1][r, sl]
                rows[1][r, sl] = (1.0 - ALPHA) * (a * nb) + ALPHA * h0v
            return c2

        lax.fori_loop(0, RB, fin_body, 0)
        pltpu.sync_copy(rows[1].at[pl.ds(0, RB)], hs_hbm.at[cid, s])


def kernel(features, edge_index, W1, b1, W2, b2):
    src = edge_index[0]
    dst = edge_index[1]
    pad = E_PAD - E
    src_p = jnp.concatenate([src, jnp.zeros((pad,), jnp.int32)])
    dst_p = jnp.concatenate([dst, jnp.full((pad,), N, jnp.int32)])
    src3 = src_p.reshape(NS, NCH, CH)
    dst3 = dst_p.reshape(NS, NCH, CH)
    dst2 = dst_p.reshape(NW, E_W)

    degp = _deg_kernel(dst2)
    h0s, y0s, nrep = _mlp_call(features, W1, b1, W2, b2, degp)
    hs = _fused_prop(y0s, h0s, nrep, src3, dst3)
    return jnp.concatenate([hs[0], hs[1]], axis=1)[:N]


# speedup vs baseline: 1.1260x; 1.0161x over previous
import functools

import jax
import jax.numpy as jnp
from jax import lax
from jax.experimental import pallas as pl
from jax.experimental.pallas import tpu as pltpu
from jax.experimental.pallas import tpu_sc as plsc

N = 10000
E = 320000
D_OUT = 64
D_HALF = D_OUT // 2
K_PROP = 10
ALPHA = 0.1

NC = 2
NS = 16
NW = NC * NS
LANES = 16

N_PAD = 10240
RPT = N_PAD // NS
CH = 128
NCH = 160
E_PAD = NS * NCH * CH
E_W = E_PAD // NW
NBUF = 8
ZR = 32
NGRP = NCH // NBUF
RB = 128
NRO = RPT // RB

_mesh = plsc.VectorSubcoreMesh(core_axis_name="c", subcore_axis_name="s")


@functools.partial(
    pl.kernel,
    out_type=jax.ShapeDtypeStruct((NC, N_PAD), jnp.float32),
    mesh=_mesh,
    compiler_params=pltpu.CompilerParams(needs_layout_passes=False),
    scratch_types=[
        pltpu.VMEM((E_W,), jnp.int32),
        pltpu.VMEM((N_PAD,), jnp.float32),
        pltpu.VMEM((RPT,), jnp.float32),
        pltpu.VMEM((RPT,), jnp.float32),
        pltpu.VMEM_SHARED((NS, N_PAD), jnp.float32),
    ],
)
def _deg_kernel(dst_hbm, degp_hbm, dst_v, hist_v, acc_v, ld_v, sh):
    cid = lax.axis_index("c")
    sid = lax.axis_index("s")
    wid = sid * NC + cid
    pltpu.sync_copy(dst_hbm.at[wid], dst_v)

    z = jnp.zeros((LANES,), jnp.float32)
    ones = jnp.ones((LANES,), jnp.float32)

    def zero_body(i, c):
        hist_v[pl.ds(i * LANES, LANES)] = z
        return c

    lax.fori_loop(0, N_PAD // LANES, zero_body, 0)

    def hist_body(i, c):
        idx = dst_v[pl.ds(i * LANES, LANES)]
        plsc.addupdate_scatter(hist_v, [idx], ones)
        return c

    lax.fori_loop(0, E_W // LANES, hist_body, 0)

    pltpu.sync_copy(hist_v, sh.at[sid])
    plsc.subcore_barrier()

    base = sid * RPT
    pltpu.sync_copy(sh.at[0, pl.ds(base, RPT)], acc_v)
    for j in range(1, NS):
        pltpu.sync_copy(sh.at[j, pl.ds(base, RPT)], ld_v)

        def add_body(i, c):
            s = pl.ds(i * LANES, LANES)
            acc_v[s] = acc_v[s] + ld_v[s]
            return c

        lax.fori_loop(0, RPT // LANES, add_body, 0)
    pltpu.sync_copy(acc_v, degp_hbm.at[cid, pl.ds(base, RPT)])


def _mlp_kernel(f_ref, w1_ref, b1_ref, w2_ref, b2_ref, degp_ref,
                h0s_ref, y0s_ref, nrep_ref):
    h = jnp.dot(f_ref[...], w1_ref[...], preferred_element_type=jnp.float32)
    h = jnp.maximum(h + b1_ref[...][None, :], 0.0)
    h = jnp.dot(h, w2_ref[...], preferred_element_type=jnp.float32)
    h = h + b2_ref[...][None, :]
    h0p = jnp.concatenate(
        [h, jnp.zeros((N_PAD - N, D_OUT), jnp.float32)], axis=0)
    deg = degp_ref[0, :] + degp_ref[1, :]
    nrm = lax.rsqrt(jnp.maximum(deg, 1.0))
    nrep_ref[...] = jnp.broadcast_to(nrm[:, None], (N_PAD, LANES))
    h0s_ref[0] = h0p[:, :D_HALF]
    h0s_ref[1] = h0p[:, D_HALF:]
    y0 = h0p * nrm[:, None]
    y0s_ref[0] = y0[:, :D_HALF]
    y0s_ref[1] = y0[:, D_HALF:]


_mlp_call = pl.pallas_call(
    _mlp_kernel,
    out_shape=(
        jax.ShapeDtypeStruct((NC, N_PAD, D_HALF), jnp.float32),
        jax.ShapeDtypeStruct((NC, N_PAD, D_HALF), jnp.float32),
        jax.ShapeDtypeStruct((N_PAD, LANES), jnp.float32),
    ),
)


@functools.partial(
    pl.kernel,
    out_type=jax.ShapeDtypeStruct((NC, N_PAD, D_HALF), jnp.float32),
    mesh=_mesh,
    compiler_params=pltpu.CompilerParams(use_tc_tiling_on_sc=False),
    scratch_types=[
        pltpu.VMEM((NCH, CH), jnp.int32),
        pltpu.VMEM((NCH, CH), jnp.int32),
        [pltpu.VMEM((CH, D_HALF), jnp.float32) for _ in range(NBUF)],
        pltpu.VMEM((ZR, D_HALF), jnp.float32),
        pltpu.VMEM((RPT, LANES), jnp.float32),
        pltpu.VMEM_SHARED((N_PAD, D_HALF), jnp.float32),
        pltpu.VMEM_SHARED((N_PAD, D_HALF), jnp.float32),
        [pltpu.SemaphoreType.DMA for _ in range(NBUF)],
        [pltpu.SemaphoreType.DMA for _ in range(NBUF)],
    ],
)
def _fused_prop(y0_hbm, h0s_hbm, nrep_hbm, src_hbm, dst_hbm, hs_hbm,
                src_v, dst_v, rows, zb, nrep_v, y_sh, agg_sh,
                gsem, ssem):
    cid = lax.axis_index("c")
    sid = lax.axis_index("s")
    base = sid * RPT
    pltpu.sync_copy(src_hbm.at[sid], src_v)
    pltpu.sync_copy(dst_hbm.at[sid], dst_v)
    pltpu.sync_copy(nrep_hbm.at[pl.ds(base, RPT)], nrep_v)

    z = jnp.zeros((LANES,), jnp.float32)

    def zb_body(r, c):
        zb[r, pl.ds(0, LANES)] = z
        zb[r, pl.ds(LANES, LANES)] = z
        return c

    lax.fori_loop(0, ZR, zb_body, 0)
    for zq in range(RPT // ZR):
        pltpu.sync_copy(zb, agg_sh.at[pl.ds(base + zq * ZR, ZR)])
    for q in range(NRO):
        s = pl.ds(base + q * RB, RB)
        pltpu.sync_copy(y0_hbm.at[cid, s], rows[0].at[pl.ds(0, RB)])
        pltpu.sync_copy(rows[0].at[pl.ds(0, RB)], y_sh.at[s])
    plsc.subcore_barrier()

    def _stream_edges():
        for b in range(NBUF):
            pltpu.async_copy(y_sh.at[src_v.at[b]], rows[b], gsem[b])

        def group_body(g, c2):
            for b in range(NBUF):
                k = g * NBUF + b
                pltpu.make_async_copy(y_sh.at[src_v.at[k]], rows[b],
                                      gsem[b]).wait()
                pltpu.async_copy(rows[b], agg_sh.at[dst_v.at[k]], ssem[b],
                                 add=True)
            for b in range(NBUF):
                k = g * NBUF + b
                pltpu.make_async_copy(rows[b], agg_sh.at[dst_v.at[k]],
                                      ssem[b]).wait()
                pltpu.async_copy(y_sh.at[src_v.at[k + NBUF]], rows[b],
                                 gsem[b])
            return c2

        lax.fori_loop(0, NGRP - 1, group_body, 0)

        for b in range(NBUF):
            k = (NGRP - 1) * NBUF + b
            pltpu.make_async_copy(y_sh.at[src_v.at[k]], rows[b],
                                  gsem[b]).wait()
            pltpu.async_copy(rows[b], agg_sh.at[dst_v.at[k]], ssem[b],
                             add=True)
        for b in range(NBUF):
            k = (NGRP - 1) * NBUF + b
            pltpu.make_async_copy(rows[b], agg_sh.at[dst_v.at[k]],
                                  ssem[b]).wait()
        plsc.subcore_barrier()

    def step_body(t, carry):
        _stream_edges()

        for q in range(NRO):
            pltpu.async_copy(y0_hbm.at[cid, pl.ds(base + q * RB, RB)],
                             rows[3 + q], gsem[q])
        for q in range(min(3, NRO)):
            pltpu.async_copy(agg_sh.at[pl.ds(base + q * RB, RB)],
                             rows[q], ssem[4 + q % 3])
        for q in range(NRO):
            s = pl.ds(base + q * RB, RB)
            aggb = rows[q % 3]
            pltpu.make_async_copy(agg_sh.at[s], aggb, ssem[4 + q % 3]).wait()
            pltpu.make_async_copy(y0_hbm.at[cid, s], rows[3 + q],
                                  gsem[q]).wait()
            y0b = rows[3 + q]

            def row_body(r, c2):
                nb = nrep_v[q * RB + r, pl.ds(0, LANES)]
                n2 = nb * nb
                for w in range(D_HALF // LANES):
                    sl = pl.ds(w * LANES, LANES)
                    a = aggb[r, sl]
                    y0v = y0b[r, sl]
                    aggb[r, sl] = (1.0 - ALPHA) * (a * n2) + ALPHA * y0v
                return c2

            lax.fori_loop(0, RB, row_body, 0)
            pltpu.sync_copy(aggb, y_sh.at[s])
            if q + 3 < NRO:
                pltpu.async_copy(agg_sh.at[pl.ds(base + (q + 3) * RB, RB)],
                                 rows[(q + 3) % 3], ssem[4 + (q + 3) % 3])
            if q > 0:
                for zq in range(RB // ZR):
                    pltpu.make_async_copy(
                        zb,
                        agg_sh.at[pl.ds(base + (q - 1) * RB + zq * ZR, ZR)],
                        ssem[zq]).wait()
            for zq in range(RB // ZR):
                pltpu.async_copy(zb,
                                 agg_sh.at[pl.ds(base + q * RB + zq * ZR,
                                                 ZR)],
                                 ssem[zq])
        for zq in range(RB // ZR):
            pltpu.make_async_copy(
                zb, agg_sh.at[pl.ds(base + (NRO - 1) * RB + zq * ZR, ZR)],
                ssem[zq]).wait()
        plsc.subcore_barrier()
        return carry

    lax.fori_loop(0, K_PROP - 1, step_body, 0)

    _stream_edges()
    for q in range(NRO):
        s = pl.ds(base + q * RB, RB)
        pltpu.sync_copy(agg_sh.at[s], rows[0].at[pl.ds(0, RB)])
        pltpu.sync_copy(h0s_hbm.at[cid, s], rows[1].at[pl.ds(0, RB)])

        def fin_body(r, c2):
            nb = nrep_v[q * RB + r, pl.ds(0, LANES)]
            for w in range(D_HALF // LANES):
                sl = pl.ds(w * LANES, LANES)
                a = rows[0][r, sl]
                h0v = rows[1][r, sl]
                rows[1][r, sl] = (1.0 - ALPHA) * (a * nb) + ALPHA * h0v
            return c2

        lax.fori_loop(0, RB, fin_body, 0)
        pltpu.sync_copy(rows[1].at[pl.ds(0, RB)], hs_hbm.at[cid, s])


def kernel(features, edge_index, W1, b1, W2, b2):
    src = edge_index[0]
    dst = edge_index[1]
    pad = E_PAD - E
    src_p = jnp.concatenate([src, jnp.zeros((pad,), jnp.int32)])
    dst_p = jnp.concatenate([dst, jnp.full((pad,), N, jnp.int32)])
    src3 = src_p.reshape(NS, NCH, CH)
    dst3 = dst_p.reshape(NS, NCH, CH)
    dst2 = dst_p.reshape(NW, E_W)

    degp = _deg_kernel(dst2)
    h0s, y0s, nrep = _mlp_call(features, W1, b1, W2, b2, degp)
    hs = _fused_prop(y0s, h0s, nrep, src3, dst3)
    return jnp.concatenate([hs[0], hs[1]], axis=1)[:N]
